# baseline (speedup 1.0000x reference)
import functools
import math

import jax
import jax.numpy as jnp
from jax import lax
from jax.experimental import pallas as pl
from jax.experimental.pallas import tpu as pltpu
from jax.experimental.pallas import tpu_sc as plsc

S = 64.0
B = 1024
C = 100000
LOG_TERM = math.log(C - 1) / S

_info = plsc.get_sparse_core_info()
_NC, _NS = _info.num_cores, _info.num_subcores
NW = _NC * _NS
TPW = B // NW

NBAND = C // 8
_BH = NBAND // NW
_EXTRA = NBAND - _BH * NW
_CHB = 4
_NFULL = _BH // _CHB

_mesh = plsc.VectorSubcoreMesh(core_axis_name="c", subcore_axis_name="s")
_params = pltpu.CompilerParams(use_tc_tiling_on_sc=True)


@functools.partial(
    pl.kernel,
    out_type=(
        jax.ShapeDtypeStruct((NW, 16), jnp.float32),
        jax.ShapeDtypeStruct((NW, 16), jnp.float32),
        jax.ShapeDtypeStruct((B, 16), jnp.float32),
    ),
    mesh=_mesh,
    scratch_types=[
        pltpu.VMEM((8 * _CHB, 1024), jnp.float32),
        pltpu.VMEM((16,), jnp.float32),
        pltpu.VMEM((8, 128), jnp.float32),
        pltpu.VMEM((TPW,), jnp.int32),
        pltpu.VMEM((TPW,), jnp.int32),
        pltpu.VMEM((TPW, 16), jnp.float32),
        pltpu.SemaphoreType.DMA,
    ],
    compiler_params=_params,
)
def _sc_scale(out_ref, lt_hbm, lband_hbm, lmod_hbm, psum_hbm, tpart_hbm,
              tv_hbm, buf, accv, tile, lb_v, lm_v, tv_buf, sem):
    wid = lax.axis_index("s") * _NC + lax.axis_index("c")
    band0 = jnp.where(wid < _EXTRA, wid * (_BH + 1),
                      _EXTRA * (_BH + 1) + (wid - _EXTRA) * _BH)
    r0w = band0 * 8

    def do_chunk(r0, nrows, acc):
        src = lt_hbm.at[pl.ds(r0, nrows), pl.ds(0, 1024)]
        dst = out_ref.at[pl.ds(r0, nrows), pl.ds(0, 1024)]
        bsl = buf.at[pl.ds(0, nrows), pl.ds(0, 1024)]
        pltpu.sync_copy(src, bsl)

        def col_body(k, a):
            for s in range(nrows):
                sl = pl.ds(k * 16, 16)
                v = buf[s, sl]
                a = a + v
                buf[s, sl] = v * S
            return a

        acc = lax.fori_loop(0, 1024 // 16, col_body, acc)
        pltpu.sync_copy(bsl, dst)
        return acc

    def chunk_body(ch, acc):
        r0 = pl.multiple_of(r0w + ch * (8 * _CHB), 8)
        return do_chunk(r0, 8 * _CHB, acc)

    acc = lax.fori_loop(0, _NFULL, chunk_body, jnp.zeros((16,), jnp.float32))
    rt = pl.multiple_of(r0w + _NFULL * (8 * _CHB), 8)
    acc = do_chunk(rt, 16, acc)

    accv[...] = acc

    @pl.when(wid < _EXTRA)
    def _():
        a2 = do_chunk(pl.multiple_of(rt + 16, 8), 8, accv[...])
        accv[...] = a2

    pltpu.sync_copy(accv, psum_hbm.at[wid])

    base = wid * TPW
    pltpu.sync_copy(lband_hbm.at[pl.ds(base, TPW)], lb_v)
    pltpu.sync_copy(lmod_hbm.at[pl.ds(base, TPW)], lm_v)
    lane = lax.iota(jnp.int32, 16)
    zero = jnp.zeros((16,), jnp.float32)
    accv[...] = zero
    for t in range(TPW):
        i = base + t
        cpos = i % 128
        aa = (cpos // 16) * 16
        ln = cpos % 16
        lbv = lb_v[pl.ds((t // 16) * 16, 16)]
        lmv = lm_v[pl.ds((t // 16) * 16, 16)]
        lb = lbv[t % 16]
        lm = lmv[t % 16]
        r0 = pl.multiple_of(lb, 8)
        c0 = (i // 128) * 128
        pltpu.sync_copy(lt_hbm.at[pl.ds(r0, 8), pl.ds(c0, 128)], tile)
        tvec = zero
        for s in range(8):
            v = tile[s, pl.ds(aa, 16)]
            flag = jnp.where(lm == s, 1.0, 0.0)
            tvec = tvec + jnp.where(lane == ln, v, zero) * flag
        tv_buf[t] = tvec
        accv[...] = accv[...] + tvec

    pltpu.sync_copy(accv, tpart_hbm.at[wid])
    pltpu.sync_copy(tv_buf, tv_hbm.at[pl.ds(base, TPW)])


@functools.partial(
    pl.kernel,
    out_type=jax.ShapeDtypeStruct((16,), jnp.float32),
    mesh=_mesh,
    scratch_types=[
        pltpu.VMEM((NW, 16), jnp.float32),
        pltpu.VMEM((NW, 16), jnp.float32),
        pltpu.VMEM((16,), jnp.float32),
        pltpu.SemaphoreType.DMA,
    ],
    compiler_params=_params,
)
def _sc_dm(psum_hbm, tpart_hbm, corr_hbm, ps_v, tp_v, cv, sem):
    wid = lax.axis_index("s") * _NC + lax.axis_index("c")

    @pl.when(wid == 0)
    def _():
        pltpu.sync_copy(psum_hbm, ps_v)
        pltpu.sync_copy(tpart_hbm, tp_v)
        sa = ps_v[0]
        st = tp_v[0]
        for w in range(1, NW):
            sa = sa + ps_v[w]
            st = st + tp_v[w]
        lane = lax.iota(jnp.int32, 16)
        for sh in (1, 2, 4, 8):
            sa = sa + sa.at[lane ^ sh].get(mode="promise_in_bounds")
            st = st + st.at[lane ^ sh].get(mode="promise_in_bounds")
        avg_p = st * (1.0 / B)
        avg_n = (sa - st) * (1.0 / (B * (C - 1)))
        cv[...] = (avg_p - avg_n - LOG_TERM) * S
        pltpu.sync_copy(cv, corr_hbm)


def kernel(logits, labels):
    labels = labels.astype(jnp.int32)
    lt = logits.T
    lband = ((labels // 8) * 8).astype(jnp.int32)
    lmod = (labels % 8).astype(jnp.int32)

    out_ref = jax.empty_ref(jax.ShapeDtypeStruct((C, B), jnp.float32))
    psum, tpart, tvrows = _sc_scale(out_ref, lt, lband, lmod)
    corr = _sc_dm(psum, tpart)

    icol = jnp.arange(B, dtype=jnp.int32)
    tvals = jnp.take_along_axis(tvrows, (icol % 16)[:, None], axis=1)[:, 0]
    vals = tvals * S - corr[0]

    out = jax.freeze(out_ref)
    out = out.at[labels, icol].set(vals)
    return out.T


# speedup vs baseline: 1.9649x; 1.9649x over previous
import functools
import math

import jax
import jax.numpy as jnp
from jax import lax
from jax.experimental import pallas as pl
from jax.experimental.pallas import tpu as pltpu
from jax.experimental.pallas import tpu_sc as plsc

S = 64.0
B = 1024
C = 100000
LOG_TERM = math.log(C - 1) / S

_info = plsc.get_sparse_core_info()
_NC, _NS = _info.num_cores, _info.num_subcores
NW = _NC * _NS
TPW = B // NW

NBAND = C // 8
_BH = NBAND // NW
_EXTRA = NBAND - _BH * NW
_CHB = 4
_NFULL = _BH // _CHB

_mesh = plsc.VectorSubcoreMesh(core_axis_name="c", subcore_axis_name="s")
_params = pltpu.CompilerParams(use_tc_tiling_on_sc=True)


@functools.partial(
    pl.kernel,
    out_type=(
        jax.ShapeDtypeStruct((NW, 16), jnp.float32),
        jax.ShapeDtypeStruct((NW, 16), jnp.float32),
    ),
    mesh=_mesh,
    scratch_types=[
        pltpu.VMEM((8 * _CHB, 1024), jnp.float32),
        pltpu.VMEM((16,), jnp.float32),
        pltpu.VMEM((8, 128), jnp.float32),
        pltpu.VMEM((TPW,), jnp.int32),
        pltpu.VMEM((TPW,), jnp.int32),
        pltpu.SemaphoreType.DMA,
    ],
    compiler_params=_params,
)
def _sc_scale(out_ref, lt_hbm, lband_hbm, lmod_hbm, psum_hbm, tpart_hbm,
              buf, accv, tile, lb_v, lm_v, sem):
    wid = lax.axis_index("s") * _NC + lax.axis_index("c")
    band0 = jnp.where(wid < _EXTRA, wid * (_BH + 1),
                      _EXTRA * (_BH + 1) + (wid - _EXTRA) * _BH)
    r0w = band0 * 8

    def do_chunk(r0, nrows, acc):
        src = lt_hbm.at[pl.ds(r0, nrows), pl.ds(0, 1024)]
        dst = out_ref.at[pl.ds(r0, nrows), pl.ds(0, 1024)]
        bsl = buf.at[pl.ds(0, nrows), pl.ds(0, 1024)]
        pltpu.sync_copy(src, bsl)

        def col_body(k, a):
            for s in range(nrows):
                sl = pl.ds(k * 16, 16)
                v = buf[s, sl]
                a = a + v
                buf[s, sl] = v * S
            return a

        acc = lax.fori_loop(0, 1024 // 16, col_body, acc)
        pltpu.sync_copy(bsl, dst)
        return acc

    def chunk_body(ch, acc):
        r0 = pl.multiple_of(r0w + ch * (8 * _CHB), 8)
        return do_chunk(r0, 8 * _CHB, acc)

    acc = lax.fori_loop(0, _NFULL, chunk_body, jnp.zeros((16,), jnp.float32))
    rt = pl.multiple_of(r0w + _NFULL * (8 * _CHB), 8)
    acc = do_chunk(rt, 16, acc)

    accv[...] = acc

    @pl.when(wid < _EXTRA)
    def _():
        a2 = do_chunk(pl.multiple_of(rt + 16, 8), 8, accv[...])
        accv[...] = a2

    pltpu.sync_copy(accv, psum_hbm.at[wid])

    base = wid * TPW
    pltpu.sync_copy(lband_hbm.at[pl.ds(base, TPW)], lb_v)
    pltpu.sync_copy(lmod_hbm.at[pl.ds(base, TPW)], lm_v)
    lane = lax.iota(jnp.int32, 16)
    zero = jnp.zeros((16,), jnp.float32)
    accv[...] = zero
    for t in range(TPW):
        i = base + t
        cpos = i % 128
        aa = (cpos // 16) * 16
        ln = cpos % 16
        lbv = lb_v[pl.ds((t // 16) * 16, 16)]
        lmv = lm_v[pl.ds((t // 16) * 16, 16)]
        lb = lbv[t % 16]
        lm = lmv[t % 16]
        r0 = pl.multiple_of(lb, 8)
        c0 = (i // 128) * 128
        pltpu.sync_copy(lt_hbm.at[pl.ds(r0, 8), pl.ds(c0, 128)], tile)
        tvec = zero
        for s in range(8):
            v = tile[s, pl.ds(aa, 16)]
            flag = jnp.where(lm == s, 1.0, 0.0)
            tvec = tvec + jnp.where(lane == ln, v, zero) * flag
        accv[...] = accv[...] + tvec

    pltpu.sync_copy(accv, tpart_hbm.at[wid])


@functools.partial(
    pl.kernel,
    out_type=jax.ShapeDtypeStruct((16,), jnp.float32),
    mesh=_mesh,
    scratch_types=[
        pltpu.VMEM((NW, 16), jnp.float32),
        pltpu.VMEM((NW, 16), jnp.float32),
        pltpu.VMEM((16,), jnp.float32),
        pltpu.SemaphoreType.DMA,
    ],
    compiler_params=_params,
)
def _sc_dm(psum_hbm, tpart_hbm, corr_hbm, ps_v, tp_v, cv, sem):
    wid = lax.axis_index("s") * _NC + lax.axis_index("c")

    @pl.when(wid == 0)
    def _():
        pltpu.sync_copy(psum_hbm, ps_v)
        pltpu.sync_copy(tpart_hbm, tp_v)
        sa = ps_v[0]
        st = tp_v[0]
        for w in range(1, NW):
            sa = sa + ps_v[w]
            st = st + tp_v[w]
        lane = lax.iota(jnp.int32, 16)
        for sh in (1, 2, 4, 8):
            sa = sa + sa.at[lane ^ sh].get(mode="promise_in_bounds")
            st = st + st.at[lane ^ sh].get(mode="promise_in_bounds")
        avg_p = st * (1.0 / B)
        avg_n = (sa - st) * (1.0 / (B * (C - 1)))
        cv[...] = (avg_p - avg_n - LOG_TERM) * S
        pltpu.sync_copy(cv, corr_hbm)


_SLOTS = 128


@functools.partial(
    pl.kernel,
    out_type=(),
    mesh=_mesh,
    scratch_types=[
        pltpu.VMEM((8, 128), jnp.float32),
        pltpu.VMEM((_SLOTS,), jnp.int32),
        pltpu.VMEM((_SLOTS,), jnp.int32),
        pltpu.VMEM((_SLOTS,), jnp.int32),
        pltpu.VMEM((16,), jnp.float32),
        pltpu.SemaphoreType.DMA,
    ],
    compiler_params=_params,
)
def _sc_fix(out_ref, tcp_hbm, tlb_hbm, tlm_hbm, corr_hbm,
            tile, cp_v, lb_v, lm_v, cv, sem):
    wid = lax.axis_index("s") * _NC + lax.axis_index("c")
    pltpu.sync_copy(tcp_hbm.at[wid], cp_v)
    pltpu.sync_copy(tlb_hbm.at[wid], lb_v)
    pltpu.sync_copy(tlm_hbm.at[wid], lm_v)
    pltpu.sync_copy(corr_hbm, cv)
    corr = cv[...]
    lane = lax.iota(jnp.int32, 16)
    zero = jnp.zeros((16,), jnp.float32)
    c0 = pl.multiple_of((wid // 4) * 128, 128)
    for t in range(_SLOTS):
        cpv = cp_v[pl.ds((t // 16) * 16, 16)]
        lbv = lb_v[pl.ds((t // 16) * 16, 16)]
        lmv = lm_v[pl.ds((t // 16) * 16, 16)]
        cp = cpv[t % 16]
        lb = lbv[t % 16]
        lm = lmv[t % 16]

        @pl.when(cp >= 0)
        def _():
            r0 = pl.multiple_of(lb, 8)
            win = out_ref.at[pl.ds(r0, 8), pl.ds(c0, 128)]
            pltpu.sync_copy(win, tile)
            aa = pl.multiple_of((cp // 16) * 16, 16)
            ln = cp - aa
            delta = jnp.where(lane == ln, corr, zero)
            for s in range(8):
                flag = jnp.where(lm == s, 1.0, 0.0)
                v = tile[s, pl.ds(aa, 16)]
                tile[s, pl.ds(aa, 16)] = v - delta * flag
            pltpu.sync_copy(tile, win)


def kernel(logits, labels):
    labels = labels.astype(jnp.int32)
    lt = logits.T
    lband = ((labels // 8) * 8).astype(jnp.int32)
    lmod = (labels % 8).astype(jnp.int32)

    out_ref = jax.empty_ref(jax.ShapeDtypeStruct((C, B), jnp.float32))
    psum, tpart = _sc_scale(out_ref, lt, lband, lmod)
    corr = _sc_dm(psum, tpart)

    icol = jnp.arange(B, dtype=jnp.int32)
    wid_t = (icol // 128) * 4 + (labels // 8) % 4
    order = jnp.argsort(wid_t)
    swid = wid_t[order]
    starts = jnp.searchsorted(swid, jnp.arange(NW, dtype=jnp.int32),
                              side="left").astype(jnp.int32)
    rank = icol - starts[swid]
    tcp = jnp.full((NW, _SLOTS), -1, jnp.int32).at[swid, rank].set(
        icol[order] % 128)
    tlb = jnp.zeros((NW, _SLOTS), jnp.int32).at[swid, rank].set(lband[order])
    tlm = jnp.zeros((NW, _SLOTS), jnp.int32).at[swid, rank].set(lmod[order])

    _sc_fix(out_ref, tcp, tlb, tlm, corr)
    return jax.freeze(out_ref).T


# speedup vs baseline: 2.1430x; 1.0906x over previous
import functools
import math

import jax
import jax.numpy as jnp
from jax import lax
from jax.experimental import pallas as pl
from jax.experimental.pallas import tpu as pltpu
from jax.experimental.pallas import tpu_sc as plsc

S = 64.0
B = 1024
C = 100000
LOG_TERM = math.log(C - 1) / S

_info = plsc.get_sparse_core_info()
_NC, _NS = _info.num_cores, _info.num_subcores
NW = _NC * _NS
TPW = B // NW

NBAND = C // 8
_BH = NBAND // NW
_EXTRA = NBAND - _BH * NW
_CHB = 4
_NFULL = _BH // _CHB

_mesh = plsc.VectorSubcoreMesh(core_axis_name="c", subcore_axis_name="s")
_params = pltpu.CompilerParams(use_tc_tiling_on_sc=True)


@functools.partial(
    pl.kernel,
    out_type=(
        jax.ShapeDtypeStruct((NW, 16), jnp.float32),
        jax.ShapeDtypeStruct((NW, 16), jnp.float32),
    ),
    mesh=_mesh,
    scratch_types=[
        pltpu.VMEM((8 * _CHB, 1024), jnp.float32),
        pltpu.VMEM((16,), jnp.float32),
        pltpu.VMEM((8, 128), jnp.float32),
        pltpu.VMEM((TPW,), jnp.int32),
        pltpu.VMEM((TPW,), jnp.int32),
        pltpu.SemaphoreType.DMA,
    ],
    compiler_params=_params,
)
def _sc_scale(out_ref, lt_hbm, lband_hbm, lmod_hbm, psum_hbm, tpart_hbm,
              buf, accv, tile, lb_v, lm_v, sem):
    wid = lax.axis_index("s") * _NC + lax.axis_index("c")
    band0 = jnp.where(wid < _EXTRA, wid * (_BH + 1),
                      _EXTRA * (_BH + 1) + (wid - _EXTRA) * _BH)
    r0w = band0 * 8

    def do_chunk(r0, nrows, acc):
        src = lt_hbm.at[pl.ds(r0, nrows), pl.ds(0, 1024)]
        dst = out_ref.at[pl.ds(r0, nrows), pl.ds(0, 1024)]
        bsl = buf.at[pl.ds(0, nrows), pl.ds(0, 1024)]
        pltpu.sync_copy(src, bsl)

        def col_body(k, a):
            for s in range(nrows):
                sl = pl.ds(k * 16, 16)
                v = buf[s, sl]
                a = a + v
                buf[s, sl] = v * S
            return a

        acc = lax.fori_loop(0, 1024 // 16, col_body, acc)
        pltpu.sync_copy(bsl, dst)
        return acc

    def chunk_body(ch, acc):
        r0 = pl.multiple_of(r0w + ch * (8 * _CHB), 8)
        return do_chunk(r0, 8 * _CHB, acc)

    acc = lax.fori_loop(0, _NFULL, chunk_body, jnp.zeros((16,), jnp.float32))
    rt = pl.multiple_of(r0w + _NFULL * (8 * _CHB), 8)
    acc = do_chunk(rt, 16, acc)

    accv[...] = acc

    @pl.when(wid < _EXTRA)
    def _():
        a2 = do_chunk(pl.multiple_of(rt + 16, 8), 8, accv[...])
        accv[...] = a2

    pltpu.sync_copy(accv, psum_hbm.at[wid])

    base = wid * TPW
    pltpu.sync_copy(lband_hbm.at[pl.ds(base, TPW)], lb_v)
    pltpu.sync_copy(lmod_hbm.at[pl.ds(base, TPW)], lm_v)
    lane = lax.iota(jnp.int32, 16)
    zero = jnp.zeros((16,), jnp.float32)
    accv[...] = zero
    for t in range(TPW):
        i = base + t
        cpos = i % 128
        aa = (cpos // 16) * 16
        ln = cpos % 16
        lbv = lb_v[pl.ds((t // 16) * 16, 16)]
        lmv = lm_v[pl.ds((t // 16) * 16, 16)]
        lb = lbv[t % 16]
        lm = lmv[t % 16]
        r0 = pl.multiple_of(lb, 8)
        c0 = (i // 128) * 128
        pltpu.sync_copy(lt_hbm.at[pl.ds(r0, 8), pl.ds(c0, 128)], tile)
        tvec = zero
        for s in range(8):
            v = tile[s, pl.ds(aa, 16)]
            flag = jnp.where(lm == s, 1.0, 0.0)
            tvec = tvec + jnp.where(lane == ln, v, zero) * flag
        accv[...] = accv[...] + tvec

    pltpu.sync_copy(accv, tpart_hbm.at[wid])


_SLOTS = 128


@functools.partial(
    pl.kernel,
    out_type=(),
    mesh=_mesh,
    scratch_types=[
        pltpu.VMEM((8, 128), jnp.float32),
        pltpu.VMEM((_SLOTS,), jnp.int32),
        pltpu.VMEM((_SLOTS,), jnp.int32),
        pltpu.VMEM((_SLOTS,), jnp.int32),
        pltpu.VMEM((NW, 16), jnp.float32),
        pltpu.VMEM((NW, 16), jnp.float32),
        pltpu.SemaphoreType.DMA,
    ],
    compiler_params=_params,
)
def _sc_fix(out_ref, tcp_hbm, tlb_hbm, tlm_hbm, psum_hbm, tpart_hbm,
            tile, cp_v, lb_v, lm_v, ps_v, tp_v, sem):
    wid = lax.axis_index("s") * _NC + lax.axis_index("c")
    pltpu.sync_copy(tcp_hbm.at[wid], cp_v)
    pltpu.sync_copy(tlb_hbm.at[wid], lb_v)
    pltpu.sync_copy(tlm_hbm.at[wid], lm_v)
    pltpu.sync_copy(psum_hbm, ps_v)
    pltpu.sync_copy(tpart_hbm, tp_v)
    lane = lax.iota(jnp.int32, 16)

    sa = ps_v[0]
    st = tp_v[0]
    for w in range(1, NW):
        sa = sa + ps_v[w]
        st = st + tp_v[w]
    for sh in (1, 2, 4, 8):
        sa = sa + sa.at[lane ^ sh].get(mode="promise_in_bounds")
        st = st + st.at[lane ^ sh].get(mode="promise_in_bounds")
    avg_p = st * (1.0 / B)
    avg_n = (sa - st) * (1.0 / (B * (C - 1)))
    corr = (avg_p - avg_n - LOG_TERM) * S
    zero = jnp.zeros((16,), jnp.float32)
    c0 = pl.multiple_of((wid // 4) * 128, 128)
    for t in range(_SLOTS):
        cpv = cp_v[pl.ds((t // 16) * 16, 16)]
        lbv = lb_v[pl.ds((t // 16) * 16, 16)]
        lmv = lm_v[pl.ds((t // 16) * 16, 16)]
        cp = cpv[t % 16]
        lb = lbv[t % 16]
        lm = lmv[t % 16]

        @pl.when(cp >= 0)
        def _():
            r0 = pl.multiple_of(lb, 8)
            win = out_ref.at[pl.ds(r0, 8), pl.ds(c0, 128)]
            pltpu.sync_copy(win, tile)
            aa = pl.multiple_of((cp // 16) * 16, 16)
            ln = cp - aa
            delta = jnp.where(lane == ln, corr, zero)
            for s in range(8):
                flag = jnp.where(lm == s, 1.0, 0.0)
                v = tile[s, pl.ds(aa, 16)]
                tile[s, pl.ds(aa, 16)] = v - delta * flag
            pltpu.sync_copy(tile, win)


def kernel(logits, labels):
    labels = labels.astype(jnp.int32)
    lt = logits.T
    lband = ((labels // 8) * 8).astype(jnp.int32)
    lmod = (labels % 8).astype(jnp.int32)

    out_ref = jax.empty_ref(jax.ShapeDtypeStruct((C, B), jnp.float32))
    psum, tpart = _sc_scale(out_ref, lt, lband, lmod)

    icol = jnp.arange(B, dtype=jnp.int32)
    wid_t = (icol // 128) * 4 + (labels // 8) % 4
    order = jnp.argsort(wid_t)
    swid = wid_t[order]
    starts = jnp.searchsorted(swid, jnp.arange(NW, dtype=jnp.int32),
                              side="left").astype(jnp.int32)
    rank = icol - starts[swid]
    tcp = jnp.full((NW, _SLOTS), -1, jnp.int32).at[swid, rank].set(
        icol[order] % 128)
    tlb = jnp.zeros((NW, _SLOTS), jnp.int32).at[swid, rank].set(lband[order])
    tlm = jnp.zeros((NW, _SLOTS), jnp.int32).at[swid, rank].set(lmod[order])

    _sc_fix(out_ref, tcp, tlb, tlm, psum, tpart)
    return jax.freeze(out_ref).T
